# baseline (speedup 1.0000x reference)
import math

import jax
import jax.numpy as jnp
from jax.experimental import pallas as pl
from jax.experimental.pallas import tpu as pltpu

_LAMDA = 0.5
_ALPHA = 0.1
_BM = 400
_BM2 = 400
_BME = 2000


def _entry_kernel(x_ref, w0t_ref, b0_ref, o_ref, obf_ref):
    h = jax.nn.relu(
        jnp.dot(x_ref[...], w0t_ref[...], preferred_element_type=jnp.float32)
        + b0_ref[...]
    )
    o_ref[...] = h
    obf_ref[...] = h.astype(jnp.bfloat16)


def _layer1_kernel(adj_ref, innerbf_ref, h0_ref, weff_ref, o_ref, adjbf_ref):
    adj_bf = adj_ref[...].astype(jnp.bfloat16)
    adjbf_ref[...] = adj_bf
    hi = jnp.dot(adj_bf, innerbf_ref[...],
                 preferred_element_type=jnp.float32)
    support = (1.0 - _ALPHA) * hi + _ALPHA * h0_ref[...]
    o_ref[...] = jax.nn.relu(
        jnp.dot(support, weff_ref[...], preferred_element_type=jnp.float32)
    ).astype(jnp.bfloat16)


def _tail_kernel(adjbf_ref, inner0_ref, h0_ref, weff_ref, o_ref, sa, sb):
    l = pl.program_id(0)
    m = pl.program_id(1)
    bm2 = o_ref.shape[0]
    adj = adjbf_ref[...]
    inner = jax.lax.switch(
        l, [lambda: inner0_ref[...], lambda: sb[...], lambda: sa[...]])
    hi = jnp.dot(adj, inner, preferred_element_type=jnp.float32)
    support = (1.0 - _ALPHA) * hi + _ALPHA * h0_ref[...]
    out = jax.nn.relu(
        jnp.dot(support, weff_ref[0], preferred_element_type=jnp.float32))
    o_ref[...] = out
    ob = out.astype(jnp.bfloat16)

    @pl.when(l == 0)
    def _():
        sb[pl.ds(m * bm2, bm2), :] = ob

    @pl.when(l == 1)
    def _():
        sa[pl.ds(m * bm2, bm2), :] = ob


def kernel(x, dia_len, topicLabel, adj, W0, b0, Wc):
    n, nfeat = x.shape
    nhid = W0.shape[0]
    nlayers = Wc.shape[0]
    bm = _BM if n % _BM == 0 else n
    grid = (n // bm,)
    bme = _BME if n % _BME == 0 else n

    h0, h0_bf = pl.pallas_call(
        _entry_kernel,
        grid=(n // bme,),
        in_specs=[
            pl.BlockSpec((bme, nfeat), lambda i: (i, 0)),
            pl.BlockSpec((nfeat, nhid), lambda i: (0, 0)),
            pl.BlockSpec((1, nhid), lambda i: (0, 0)),
        ],
        out_specs=[
            pl.BlockSpec((bme, nhid), lambda i: (i, 0)),
            pl.BlockSpec((bme, nhid), lambda i: (i, 0)),
        ],
        out_shape=[
            jax.ShapeDtypeStruct((n, nhid), jnp.float32),
            jax.ShapeDtypeStruct((n, nhid), jnp.bfloat16),
        ],
        compiler_params=pltpu.CompilerParams(
            dimension_semantics=("parallel",)),
    )(x, W0.T, b0.reshape(1, nhid))

    eye = jnp.eye(nhid, dtype=jnp.float32)

    def w_eff(i):
        theta = math.log(_LAMDA / (i + 1) + 1.0)
        return theta * Wc[i] + (1.0 - theta) * eye

    inner_bf, adj_bf = pl.pallas_call(
        _layer1_kernel,
        grid=grid,
        in_specs=[
            pl.BlockSpec((bm, n), lambda i: (i, 0)),
            pl.BlockSpec((n, nhid), lambda i: (0, 0)),
            pl.BlockSpec((bm, nhid), lambda i: (i, 0)),
            pl.BlockSpec((nhid, nhid), lambda i: (0, 0)),
        ],
        out_specs=[
            pl.BlockSpec((bm, nhid), lambda i: (i, 0)),
            pl.BlockSpec((bm, n), lambda i: (i, 0)),
        ],
        out_shape=[
            jax.ShapeDtypeStruct((n, nhid), jnp.bfloat16),
            jax.ShapeDtypeStruct((n, n), jnp.bfloat16),
        ],
        compiler_params=pltpu.CompilerParams(
            dimension_semantics=("parallel",)),
    )(adj, h0_bf, h0, w_eff(0))

    bm2 = _BM2 if n % _BM2 == 0 else bm
    w_eff_tail = jnp.stack([w_eff(i) for i in range(1, nlayers)])
    out = pl.pallas_call(
        _tail_kernel,
        grid=(nlayers - 1, n // bm2),
        in_specs=[
            pl.BlockSpec((bm2, n), lambda l, m: (m, 0)),
            pl.BlockSpec((n, nhid), lambda l, m: (0, 0)),
            pl.BlockSpec((bm2, nhid), lambda l, m: (m, 0)),
            pl.BlockSpec((1, nhid, nhid), lambda l, m: (l, 0, 0)),
        ],
        out_specs=pl.BlockSpec((bm2, nhid), lambda l, m: (m, 0)),
        out_shape=jax.ShapeDtypeStruct((n, nhid), jnp.float32),
        scratch_shapes=[
            pltpu.VMEM((n, nhid), jnp.bfloat16),
            pltpu.VMEM((n, nhid), jnp.bfloat16),
        ],
        compiler_params=pltpu.CompilerParams(
            dimension_semantics=("arbitrary", "arbitrary")),
    )(adj_bf, inner_bf, h0, w_eff_tail)
    return out


# speedup vs baseline: 1.4876x; 1.4876x over previous
import math

import jax
import jax.numpy as jnp
from jax.experimental import pallas as pl
from jax.experimental.pallas import tpu as pltpu

_LAMDA = 0.5
_ALPHA = 0.1
_BM = 400
_BM2 = 400
_BME = 2000


def _entry_kernel(x_ref, w0t_ref, b0_ref, o_ref, obf_ref):
    h = jax.nn.relu(
        jnp.dot(x_ref[...], w0t_ref[...], preferred_element_type=jnp.float32)
        + b0_ref[...]
    )
    o_ref[...] = h
    obf_ref[...] = h.astype(jnp.bfloat16)


def _layer1_kernel(adj_ref, innerbf_ref, h0_ref, weff_ref, o_ref, adjbf_ref):
    adj_bf = adj_ref[...].astype(jnp.bfloat16)
    adjbf_ref[...] = adj_bf
    hi = jnp.dot(adj_bf, innerbf_ref[...],
                 preferred_element_type=jnp.float32)
    support = (1.0 - _ALPHA) * hi + _ALPHA * h0_ref[...]
    o_ref[...] = jax.nn.relu(
        jnp.dot(support, weff_ref[...], preferred_element_type=jnp.float32)
    ).astype(jnp.bfloat16)


def _layer_kernel(adja_ref, adjb_ref, innerbf_ref, h0_ref, weff_ref, o_ref):
    inner = innerbf_ref[...]
    hi = jnp.concatenate(
        [jnp.dot(adja_ref[...], inner, preferred_element_type=jnp.float32),
         jnp.dot(adjb_ref[...], inner, preferred_element_type=jnp.float32)],
        axis=0)
    support = (1.0 - _ALPHA) * hi + _ALPHA * h0_ref[...]
    out = jax.nn.relu(
        jnp.dot(support, weff_ref[...], preferred_element_type=jnp.float32))
    o_ref[...] = out.astype(o_ref.dtype)


def kernel(x, dia_len, topicLabel, adj, W0, b0, Wc):
    n, nfeat = x.shape
    nhid = W0.shape[0]
    nlayers = Wc.shape[0]
    bm = _BM if n % _BM == 0 else n
    grid = (n // bm,)
    bme = _BME if n % _BME == 0 else n

    h0, h0_bf = pl.pallas_call(
        _entry_kernel,
        grid=(n // bme,),
        in_specs=[
            pl.BlockSpec((bme, nfeat), lambda i: (i, 0)),
            pl.BlockSpec((nfeat, nhid), lambda i: (0, 0)),
            pl.BlockSpec((1, nhid), lambda i: (0, 0)),
        ],
        out_specs=[
            pl.BlockSpec((bme, nhid), lambda i: (i, 0)),
            pl.BlockSpec((bme, nhid), lambda i: (i, 0)),
        ],
        out_shape=[
            jax.ShapeDtypeStruct((n, nhid), jnp.float32),
            jax.ShapeDtypeStruct((n, nhid), jnp.bfloat16),
        ],
        compiler_params=pltpu.CompilerParams(
            dimension_semantics=("parallel",)),
    )(x, W0.T, b0.reshape(1, nhid))

    eye = jnp.eye(nhid, dtype=jnp.float32)

    def w_eff(i):
        theta = math.log(_LAMDA / (i + 1) + 1.0)
        return theta * Wc[i] + (1.0 - theta) * eye

    inner_bf, adj_bf = pl.pallas_call(
        _layer1_kernel,
        grid=grid,
        in_specs=[
            pl.BlockSpec((bm, n), lambda i: (i, 0)),
            pl.BlockSpec((n, nhid), lambda i: (0, 0)),
            pl.BlockSpec((bm, nhid), lambda i: (i, 0)),
            pl.BlockSpec((nhid, nhid), lambda i: (0, 0)),
        ],
        out_specs=[
            pl.BlockSpec((bm, nhid), lambda i: (i, 0)),
            pl.BlockSpec((bm, n), lambda i: (i, 0)),
        ],
        out_shape=[
            jax.ShapeDtypeStruct((n, nhid), jnp.bfloat16),
            jax.ShapeDtypeStruct((n, n), jnp.bfloat16),
        ],
        compiler_params=pltpu.CompilerParams(
            dimension_semantics=("parallel",)),
    )(adj, h0_bf, h0, w_eff(0))

    half = _BM2 // 2 if n % _BM2 == 0 else n // 2
    grid2 = (n // (2 * half),)
    for i in range(1, nlayers):
        out_dtype = jnp.float32 if i == nlayers - 1 else jnp.bfloat16
        inner_bf = pl.pallas_call(
            _layer_kernel,
            grid=grid2,
            in_specs=[
                pl.BlockSpec((half, n), lambda i: (2 * i, 0)),
                pl.BlockSpec((half, n), lambda i: (2 * i + 1, 0)),
                pl.BlockSpec((n, nhid), lambda i: (0, 0)),
                pl.BlockSpec((2 * half, nhid), lambda i: (i, 0)),
                pl.BlockSpec((nhid, nhid), lambda i: (0, 0)),
            ],
            out_specs=pl.BlockSpec((2 * half, nhid), lambda i: (i, 0)),
            out_shape=jax.ShapeDtypeStruct((n, nhid), out_dtype),
            compiler_params=pltpu.CompilerParams(
                dimension_semantics=("parallel",)),
        )(adj_bf, adj_bf, inner_bf, h0, w_eff(i))
    return inner_bf


# speedup vs baseline: 1.5108x; 1.0156x over previous
import math

import jax
import jax.numpy as jnp
from jax.experimental import pallas as pl
from jax.experimental.pallas import tpu as pltpu

_LAMDA = 0.5
_ALPHA = 0.1
_BM = 200
_BM2 = 1000
_BME = 2000


def _entry_kernel(x_ref, w0t_ref, b0_ref, o_ref, obf_ref):
    h = jax.nn.relu(
        jnp.dot(x_ref[...], w0t_ref[...], preferred_element_type=jnp.float32)
        + b0_ref[...]
    )
    o_ref[...] = h
    obf_ref[...] = h.astype(jnp.bfloat16)


def _layer1_kernel(adj_ref, innerbf_ref, h0_ref, weff_ref, o_ref, adjbf_ref):
    adj_bf = adj_ref[...].astype(jnp.bfloat16)
    adjbf_ref[...] = adj_bf[None]
    hi = jnp.dot(adj_bf, innerbf_ref[...],
                 preferred_element_type=jnp.float32)
    support = (1.0 - _ALPHA) * hi + _ALPHA * h0_ref[...]
    o_ref[...] = jax.nn.relu(
        jnp.dot(support, weff_ref[...], preferred_element_type=jnp.float32)
    ).astype(jnp.bfloat16)


def _layer_kernel(adjbf_ref, innerbf_ref, h0_ref, weff_ref, o_ref):
    hi = jnp.dot(adjbf_ref[0], innerbf_ref[...],
                 preferred_element_type=jnp.float32)
    support = (1.0 - _ALPHA) * hi + _ALPHA * h0_ref[...]
    out = jax.nn.relu(
        jnp.dot(support, weff_ref[...], preferred_element_type=jnp.float32))
    o_ref[...] = out.astype(o_ref.dtype)


def kernel(x, dia_len, topicLabel, adj, W0, b0, Wc):
    n, nfeat = x.shape
    nhid = W0.shape[0]
    nlayers = Wc.shape[0]
    bm = _BM if n % _BM == 0 else n
    grid = (n // bm,)
    bme = _BME if n % _BME == 0 else n

    h0, h0_bf = pl.pallas_call(
        _entry_kernel,
        grid=(n // bme,),
        in_specs=[
            pl.BlockSpec((bme, nfeat), lambda i: (i, 0)),
            pl.BlockSpec((nfeat, nhid), lambda i: (0, 0)),
            pl.BlockSpec((1, nhid), lambda i: (0, 0)),
        ],
        out_specs=[
            pl.BlockSpec((bme, nhid), lambda i: (i, 0)),
            pl.BlockSpec((bme, nhid), lambda i: (i, 0)),
        ],
        out_shape=[
            jax.ShapeDtypeStruct((n, nhid), jnp.float32),
            jax.ShapeDtypeStruct((n, nhid), jnp.bfloat16),
        ],
        compiler_params=pltpu.CompilerParams(
            dimension_semantics=("parallel",)),
    )(x, W0.T, b0.reshape(1, nhid))

    eye = jnp.eye(nhid, dtype=jnp.float32)

    def w_eff(i):
        theta = math.log(_LAMDA / (i + 1) + 1.0)
        return theta * Wc[i] + (1.0 - theta) * eye

    slab = _BM2 if n % _BM2 == 0 else bm
    per_slab = slab // bm
    inner_bf, adj_bf = pl.pallas_call(
        _layer1_kernel,
        grid=grid,
        in_specs=[
            pl.BlockSpec((bm, n), lambda i: (i, 0)),
            pl.BlockSpec((n, nhid), lambda i: (0, 0)),
            pl.BlockSpec((bm, nhid), lambda i: (i, 0)),
            pl.BlockSpec((nhid, nhid), lambda i: (0, 0)),
        ],
        out_specs=[
            pl.BlockSpec((bm, nhid), lambda i: (i, 0)),
            pl.BlockSpec((1, bm, n),
                         lambda i: (i // per_slab, i % per_slab, 0)),
        ],
        out_shape=[
            jax.ShapeDtypeStruct((n, nhid), jnp.bfloat16),
            jax.ShapeDtypeStruct((n // slab, slab, n), jnp.bfloat16),
        ],
        compiler_params=pltpu.CompilerParams(
            dimension_semantics=("parallel",)),
    )(adj, h0_bf, h0, w_eff(0))

    grid2 = (n // slab,)
    for i in range(1, nlayers):
        out_dtype = jnp.float32 if i == nlayers - 1 else jnp.bfloat16
        inner_bf = pl.pallas_call(
            _layer_kernel,
            grid=grid2,
            in_specs=[
                pl.BlockSpec((1, slab, n), lambda i: (i, 0, 0)),
                pl.BlockSpec((n, nhid), lambda i: (0, 0)),
                pl.BlockSpec((slab, nhid), lambda i: (i, 0)),
                pl.BlockSpec((nhid, nhid), lambda i: (0, 0)),
            ],
            out_specs=pl.BlockSpec((slab, nhid), lambda i: (i, 0)),
            out_shape=jax.ShapeDtypeStruct((n, nhid), out_dtype),
            compiler_params=pltpu.CompilerParams(
                dimension_semantics=("parallel",)),
        )(adj_bf, inner_bf, h0, w_eff(i))
    return inner_bf


# speedup vs baseline: 1.5193x; 1.0056x over previous
import math

import jax
import jax.numpy as jnp
from jax.experimental import pallas as pl
from jax.experimental.pallas import tpu as pltpu

_LAMDA = 0.5
_ALPHA = 0.1
_BM = 200
_BM2 = 1000
_BME = 2000


def _entry_kernel(x_ref, w0t_ref, b0_ref, o_ref, obf_ref):
    h = jax.nn.relu(
        jnp.dot(x_ref[...], w0t_ref[...], preferred_element_type=jnp.float32)
        + b0_ref[...]
    )
    o_ref[...] = h
    obf_ref[...] = h.astype(jnp.bfloat16)


def _layer1_kernel(adj_ref, innerbf_ref, h0_ref, weff_ref, o_ref, adjbf_ref):
    adj_bf = adj_ref[...].astype(jnp.bfloat16)
    adjbf_ref[...] = adj_bf[None]
    hi = jnp.dot(adj_bf, innerbf_ref[...],
                 preferred_element_type=jnp.float32)
    support = (1.0 - _ALPHA) * hi + _ALPHA * h0_ref[...]
    o_ref[...] = jax.nn.relu(
        jnp.dot(support, weff_ref[...], preferred_element_type=jnp.float32)
    ).astype(jnp.bfloat16)


def _layer_kernel(adjbf_ref, innerbf_ref, h0_ref, weff_ref, o_ref):
    hi = jnp.dot(adjbf_ref[0], innerbf_ref[...],
                 preferred_element_type=jnp.float32)
    support = (1.0 - _ALPHA) * hi + _ALPHA * h0_ref[...].astype(jnp.float32)
    out = jax.nn.relu(
        jnp.dot(support, weff_ref[...], preferred_element_type=jnp.float32))
    o_ref[...] = out.astype(o_ref.dtype)


def kernel(x, dia_len, topicLabel, adj, W0, b0, Wc):
    n, nfeat = x.shape
    nhid = W0.shape[0]
    nlayers = Wc.shape[0]
    bm = _BM if n % _BM == 0 else n
    grid = (n // bm,)
    bme = _BME if n % _BME == 0 else n

    h0, h0_bf = pl.pallas_call(
        _entry_kernel,
        grid=(n // bme,),
        in_specs=[
            pl.BlockSpec((bme, nfeat), lambda i: (i, 0)),
            pl.BlockSpec((nfeat, nhid), lambda i: (0, 0)),
            pl.BlockSpec((1, nhid), lambda i: (0, 0)),
        ],
        out_specs=[
            pl.BlockSpec((bme, nhid), lambda i: (i, 0)),
            pl.BlockSpec((bme, nhid), lambda i: (i, 0)),
        ],
        out_shape=[
            jax.ShapeDtypeStruct((n, nhid), jnp.float32),
            jax.ShapeDtypeStruct((n, nhid), jnp.bfloat16),
        ],
        compiler_params=pltpu.CompilerParams(
            dimension_semantics=("parallel",)),
    )(x, W0.T, b0.reshape(1, nhid))

    eye = jnp.eye(nhid, dtype=jnp.float32)

    def w_eff(i):
        theta = math.log(_LAMDA / (i + 1) + 1.0)
        return theta * Wc[i] + (1.0 - theta) * eye

    slab = _BM2 if n % _BM2 == 0 else bm
    per_slab = slab // bm
    inner_bf, adj_bf = pl.pallas_call(
        _layer1_kernel,
        grid=grid,
        in_specs=[
            pl.BlockSpec((bm, n), lambda i: (i, 0)),
            pl.BlockSpec((n, nhid), lambda i: (0, 0)),
            pl.BlockSpec((bm, nhid), lambda i: (i, 0)),
            pl.BlockSpec((nhid, nhid), lambda i: (0, 0)),
        ],
        out_specs=[
            pl.BlockSpec((bm, nhid), lambda i: (i, 0)),
            pl.BlockSpec((1, bm, n),
                         lambda i: (i // per_slab, i % per_slab, 0)),
        ],
        out_shape=[
            jax.ShapeDtypeStruct((n, nhid), jnp.bfloat16),
            jax.ShapeDtypeStruct((n // slab, slab, n), jnp.bfloat16),
        ],
        compiler_params=pltpu.CompilerParams(
            dimension_semantics=("parallel",)),
    )(adj, h0_bf, h0, w_eff(0))

    grid2 = (n // slab,)
    for i in range(1, nlayers):
        out_dtype = jnp.float32 if i == nlayers - 1 else jnp.bfloat16
        inner_bf = pl.pallas_call(
            _layer_kernel,
            grid=grid2,
            in_specs=[
                pl.BlockSpec((1, slab, n), lambda i: (i, 0, 0)),
                pl.BlockSpec((n, nhid), lambda i: (0, 0)),
                pl.BlockSpec((slab, nhid), lambda i: (i, 0)),
                pl.BlockSpec((nhid, nhid), lambda i: (0, 0)),
            ],
            out_specs=pl.BlockSpec((slab, nhid), lambda i: (i, 0)),
            out_shape=jax.ShapeDtypeStruct((n, nhid), out_dtype),
            compiler_params=pltpu.CompilerParams(
                dimension_semantics=("parallel",)),
        )(adj_bf, inner_bf, h0_bf, w_eff(i))
    return inner_bf
